# baseline (speedup 1.0000x reference)
import functools

import jax
import jax.numpy as jnp
from jax import lax
from jax.experimental import pallas as pl
from jax.experimental.pallas import tpu as pltpu
from jax.experimental.pallas import tpu_sc as plsc

N = 10000
E = 320000
DN = 128
DE = 16
DG = 32
H = 32
PK = 4
E4 = E // PK
CH = 125
EC = E // CH
SLAB = 16
SPR = SLAB * CH // PK
NW = 32
EPW = E // NW
CPW = EPW // CH
NSLAB = CPW // SLAB
NS = 16
NP = 10240
NPT = NP // NS
RB_E = 4000
RB_N = 2000

f32 = jnp.float32


def _softplus(x):
    return jnp.maximum(x, 0.0) + jnp.log1p(jnp.exp(-jnp.abs(x)))


def _tc1_body(x_ref, w_ref, ps_ref, pd_ref):
    p = jnp.dot(x_ref[...], w_ref[...], preferred_element_type=f32)
    ps_ref[...] = p[:, :H]
    pd_ref[...] = p[:, H:]


def _project(x, w_sd):
    return pl.pallas_call(
        _tc1_body,
        grid=(1,),
        in_specs=[
            pl.BlockSpec((N, DN), lambda i: (0, 0)),
            pl.BlockSpec((DN, 2 * H), lambda i: (0, 0)),
        ],
        out_specs=[
            pl.BlockSpec((N, H), lambda i: (0, 0)),
            pl.BlockSpec((N, H), lambda i: (0, 0)),
        ],
        out_shape=[
            jax.ShapeDtypeStruct((N, H), f32),
            jax.ShapeDtypeStruct((N, H), f32),
        ],
    )(x, w_sd)


_sc_mesh = plsc.VectorSubcoreMesh(core_axis_name="c", subcore_axis_name="s")


@functools.partial(
    pl.kernel,
    out_type=[
        jax.ShapeDtypeStruct((EC, CH, H), f32),
        jax.ShapeDtypeStruct((EC, CH, H), f32),
    ],
    mesh=_sc_mesh,
    scratch_types=[
        pltpu.VMEM((SLAB, CH), jnp.int32),
        pltpu.VMEM((SLAB, CH), jnp.int32),
        pltpu.VMEM((SLAB, CH, H), f32),
        pltpu.SemaphoreType.DMA,
    ],
    compiler_params=pltpu.CompilerParams(use_tc_tiling_on_sc=False),
)
def _sc_gather(ps_hbm, pd_hbm, src_hbm, dst_hbm, gs_hbm, gd_hbm,
               sidx, didx, rows, sem):
    wid = lax.axis_index("s") * 2 + lax.axis_index("c")

    def slab_body(sl, carry):
        row0 = wid * CPW + sl * SLAB
        pltpu.sync_copy(src_hbm.at[pl.ds(row0, SLAB)], sidx)
        descs = [
            pltpu.async_copy(ps_hbm.at[sidx.at[j]], rows.at[j], sem)
            for j in range(SLAB)
        ]
        for d in descs:
            d.wait()
        pltpu.sync_copy(rows, gs_hbm.at[pl.ds(row0, SLAB)])
        pltpu.sync_copy(dst_hbm.at[pl.ds(row0, SLAB)], didx)
        descs = [
            pltpu.async_copy(pd_hbm.at[didx.at[j]], rows.at[j], sem)
            for j in range(SLAB)
        ]
        for d in descs:
            d.wait()
        pltpu.sync_copy(rows, gd_hbm.at[pl.ds(row0, SLAB)])
        return carry

    lax.fori_loop(0, NSLAB, slab_body, 0)


def _tc2_body(gs_ref, gd_ref, ea_ref, g4_ref, wg4_ref, b1_ref, wd1_ref,
              wd2_ref, b2_ref, wd3_ref, b3_ref, he_ref, sum_ref):
    i = pl.program_id(0)
    c128 = jnp.dot(g4_ref[...], wg4_ref[...], preferred_element_type=f32) + b1_ref[...]
    z = (gs_ref[...] + gd_ref[...]
         + jnp.dot(ea_ref[...], wd1_ref[...], preferred_element_type=f32)
         + c128)
    h1 = _softplus(z)
    h2 = _softplus(jnp.dot(h1, wd2_ref[...], preferred_element_type=f32) + b2_ref[...])
    he = jnp.dot(h2, wd3_ref[...], preferred_element_type=f32) + b3_ref[...]
    he_ref[...] = he
    bsum = jnp.sum(he, axis=0, keepdims=True)

    @pl.when(i == 0)
    def _():
        sum_ref[...] = bsum

    @pl.when(i != 0)
    def _():
        sum_ref[...] = sum_ref[...] + bsum


def _edge_mlp(gs4, gd4, ea4, g4, wg4, b1_4, wd1, wd2, b2_4, wd3, b3_4):
    grid = (E4 // RB_E,)
    full = lambda shape: pl.BlockSpec(shape, lambda i: (0, 0))
    return pl.pallas_call(
        _tc2_body,
        grid=grid,
        in_specs=[
            pl.BlockSpec((RB_E, PK * H), lambda i: (i, 0)),
            pl.BlockSpec((RB_E, PK * H), lambda i: (i, 0)),
            pl.BlockSpec((RB_E, PK * DE), lambda i: (i, 0)),
            full((1, PK * DG)),
            full((PK * DG, PK * H)),
            full((1, PK * H)),
            full((PK * DE, PK * H)),
            full((PK * H, PK * H)),
            full((1, PK * H)),
            full((PK * H, PK * H)),
            full((1, PK * H)),
        ],
        out_specs=[
            pl.BlockSpec((RB_E, PK * H), lambda i: (i, 0)),
            pl.BlockSpec((1, PK * H), lambda i: (0, 0)),
        ],
        out_shape=[
            jax.ShapeDtypeStruct((E4, PK * H), f32),
            jax.ShapeDtypeStruct((1, PK * H), f32),
        ],
    )(gs4, gd4, ea4, g4, wg4, b1_4, wd1, wd2, b2_4, wd3, b3_4)


@functools.partial(
    pl.kernel,
    out_type=[
        jax.ShapeDtypeStruct((2, NP, H), f32),
        jax.ShapeDtypeStruct((2, NP, DE), f32),
        jax.ShapeDtypeStruct((EC, CH, H), f32),
    ],
    mesh=_sc_mesh,
    scratch_types=[
        pltpu.VMEM_SHARED((NP, H), f32),
        pltpu.VMEM_SHARED((NP, DE), f32),
        pltpu.VMEM((SLAB, CH), jnp.int32),
        pltpu.VMEM((SLAB, CH, H), f32),
        pltpu.VMEM((CH, DE), f32),
        pltpu.VMEM((NPT, H), f32),
        pltpu.VMEM((NPT, DE), f32),
        pltpu.SemaphoreType.DMA,
        pltpu.SemaphoreType.DMA,
    ],
    compiler_params=pltpu.CompilerParams(use_tc_tiling_on_sc=False),
)
def _sc_scatter(he_hbm, dst_hbm, z32_hbm, z16_hbm, ones_hbm, acc_hbm, cnt_hbm,
                he_out, sh_acc, sh_cnt, idx, rows, ones, stage, stage16,
                sem, sem2):
    cid = lax.axis_index("c")
    sid = lax.axis_index("s")
    wid = sid * 2 + cid

    pltpu.sync_copy(z32_hbm.at[pl.ds(sid * NPT, NPT)], stage)
    pltpu.sync_copy(stage, sh_acc.at[pl.ds(sid * NPT, NPT)])
    pltpu.sync_copy(z16_hbm.at[pl.ds(sid * NPT, NPT)], stage16)
    pltpu.sync_copy(stage16, sh_cnt.at[pl.ds(sid * NPT, NPT)])
    pltpu.sync_copy(ones_hbm, ones)
    plsc.subcore_barrier()

    def slab_body(sl, carry):
        row0 = wid * CPW + sl * SLAB
        pltpu.sync_copy(dst_hbm.at[pl.ds(row0, SLAB)], idx)
        pltpu.sync_copy(he_hbm.at[pl.ds(row0, SLAB)], rows)
        descs = [
            pltpu.async_copy(rows.at[j], sh_acc.at[idx.at[j]], sem, add=True)
            for j in range(SLAB)
        ]
        descs2 = [
            pltpu.async_copy(ones, sh_cnt.at[idx.at[j]], sem2, add=True)
            for j in range(SLAB)
        ]
        pltpu.sync_copy(rows, he_out.at[pl.ds(row0, SLAB)])
        for d in descs:
            d.wait()
        for d in descs2:
            d.wait()
        return carry

    lax.fori_loop(0, NSLAB, slab_body, 0)
    plsc.subcore_barrier()

    pltpu.sync_copy(sh_acc.at[pl.ds(sid * NPT, NPT)], stage)
    pltpu.sync_copy(stage, acc_hbm.at[cid, pl.ds(sid * NPT, NPT)])
    pltpu.sync_copy(sh_cnt.at[pl.ds(sid * NPT, NPT)], stage16)
    pltpu.sync_copy(stage16, cnt_hbm.at[cid, pl.ds(sid * NPT, NPT)])


def _tc3_body(x_ref, acc_ref, cnt_ref, she_ref, g_ref,
              wnx_ref, wne_ref, wng_ref, bn1_ref, wn2_ref, bn2_ref,
              wn3_ref, bn3_ref,
              wg1e_ref, wg1n_ref, wg1g_ref, bg1_ref, wg2_ref, bg2_ref,
              wg3_ref, bg3_ref,
              hn_ref, hu_ref, ssum):
    i = pl.program_id(0)
    cnt = cnt_ref[0, :, 0:1] + cnt_ref[1, :, 0:1]
    e_mean = (acc_ref[0] + acc_ref[1]) / jnp.maximum(cnt, 1.0)
    grow = jnp.dot(g_ref[...], wng_ref[...], preferred_element_type=f32)
    h1 = _softplus(jnp.dot(x_ref[...], wnx_ref[...], preferred_element_type=f32)
                   + jnp.dot(e_mean, wne_ref[...], preferred_element_type=f32)
                   + grow + bn1_ref[...])
    h2 = _softplus(jnp.dot(h1, wn2_ref[...], preferred_element_type=f32) + bn2_ref[...])
    hn = jnp.dot(h2, wn3_ref[...], preferred_element_type=f32) + bn3_ref[...]
    hn_ref[...] = hn
    bsum = jnp.sum(hn, axis=0, keepdims=True)

    @pl.when(i == 0)
    def _():
        ssum[...] = bsum

    @pl.when(i != 0)
    def _():
        ssum[...] = ssum[...] + bsum

    e_mean_g = jnp.sum(she_ref[...], axis=0, keepdims=True) * (1.0 / E)
    n_mean_g = ssum[...] * (1.0 / N)
    zg = (jnp.dot(e_mean_g, wg1e_ref[...], preferred_element_type=f32)
          + jnp.dot(n_mean_g, wg1n_ref[...], preferred_element_type=f32)
          + jnp.dot(g_ref[...], wg1g_ref[...], preferred_element_type=f32)
          + bg1_ref[...])
    hg1 = _softplus(zg)
    hg2 = _softplus(jnp.dot(hg1, wg2_ref[...], preferred_element_type=f32) + bg2_ref[...])
    hu_ref[...] = jnp.dot(hg2, wg3_ref[...], preferred_element_type=f32) + bg3_ref[...]


def _node_global(x, acc, cnt, she, g, wnx, wne, wng, bn1, wn2, bn2, wn3, bn3,
                 wg1e, wg1n, wg1g, bg1, wg2, bg2, wg3, bg3):
    grid = (N // RB_N,)
    full = lambda shape: pl.BlockSpec(shape, lambda i: tuple(0 for _ in shape))
    return pl.pallas_call(
        _tc3_body,
        grid=grid,
        in_specs=[
            pl.BlockSpec((RB_N, DN), lambda i: (i, 0)),
            pl.BlockSpec((2, RB_N, H), lambda i: (0, i, 0)),
            pl.BlockSpec((2, RB_N, DE), lambda i: (0, i, 0)),
            full((PK, H)),
            full((1, DG)),
            full((DN, H)),
            full((H, H)),
            full((DG, H)),
            full((1, H)),
            full((H, H)),
            full((1, H)),
            full((H, H)),
            full((1, H)),
            full((H, DG)),
            full((H, DG)),
            full((DG, DG)),
            full((1, DG)),
            full((DG, DG)),
            full((1, DG)),
            full((DG, DG)),
            full((1, DG)),
        ],
        out_specs=[
            pl.BlockSpec((RB_N, H), lambda i: (i, 0)),
            pl.BlockSpec((1, DG), lambda i: (0, 0)),
        ],
        out_shape=[
            jax.ShapeDtypeStruct((N, H), f32),
            jax.ShapeDtypeStruct((1, DG), f32),
        ],
        scratch_shapes=[pltpu.VMEM((1, H), f32)],
    )(x, acc, cnt, she, g, wnx, wne, wng, bn1, wn2, bn2, wn3, bn3,
      wg1e, wg1n, wg1g, bg1, wg2, bg2, wg3, bg3)


def kernel(edge_index, x, edge_attr, global_feats, batch,
           We1, be1, We2, be2, We3, be3,
           Wn1, bn1, Wn2, bn2, Wn3, bn3,
           Wg1, bg1, Wg2, bg2, Wg3, bg3):
    w_s = We1[:DN]
    w_d = We1[DN:2 * DN]
    w_e = We1[2 * DN:2 * DN + DE]
    w_g = We1[2 * DN + DE:]
    eye = jnp.eye(PK, dtype=f32)
    wd1 = jnp.kron(eye, w_e)
    wg4 = jnp.kron(eye, w_g)
    wd2 = jnp.kron(eye, We2)
    wd3 = jnp.kron(eye, We3)
    b1_4 = jnp.tile(be1, PK)[None, :]
    b2_4 = jnp.tile(be2, PK)[None, :]
    b3_4 = jnp.tile(be3, PK)[None, :]
    g4 = jnp.tile(global_feats, (1, PK))

    src2d = edge_index[0].reshape(EC, CH)
    dst2d = edge_index[1].reshape(EC, CH)
    ea4 = edge_attr.reshape(E4, PK * DE)
    z32 = jnp.zeros((NP, H), f32)
    z16 = jnp.zeros((NP, DE), f32)
    ones16 = jnp.ones((CH, DE), f32)

    pre_s, pre_d = _project(x, jnp.concatenate([w_s, w_d], axis=1))

    gs, gd = _sc_gather(pre_s, pre_d, src2d, dst2d)

    he4, sum128 = _edge_mlp(gs.reshape(E4, PK * H), gd.reshape(E4, PK * H),
                            ea4, g4, wg4, b1_4, wd1, wd2, b2_4, wd3, b3_4)

    acc, cnt, he_lin = _sc_scatter(he4.reshape(EC, CH, H), dst2d,
                                   z32, z16, ones16)
    h_e = he_lin.reshape(E, H)

    h_n, h_u = _node_global(
        x, acc, cnt, sum128.reshape(PK, H), global_feats,
        Wn1[:DN], Wn1[DN:DN + H], Wn1[DN + H:], bn1[None, :],
        Wn2, bn2[None, :], Wn3, bn3[None, :],
        Wg1[:H], Wg1[H:2 * H], Wg1[2 * H:], bg1[None, :],
        Wg2, bg2[None, :], Wg3, bg3[None, :])

    return (h_e, h_n, h_u)


# speedup vs baseline: 1.2234x; 1.2234x over previous
import functools

import jax
import jax.numpy as jnp
from jax import lax
from jax.experimental import pallas as pl
from jax.experimental.pallas import tpu as pltpu
from jax.experimental.pallas import tpu_sc as plsc

N = 10000
E = 320000
DN = 128
DE = 16
DG = 32
H = 32
PK = 4
E4 = E // PK
CH = 125
EC = E // CH
SLAB = 16
NW = 32
EPW = E // NW
CPW = EPW // CH
NSLAB = CPW // SLAB
NS = 16
NP = 10240
NPT = NP // NS
RB_E = 4000
RB_N = 2000

f32 = jnp.float32


def _softplus(x):
    return jnp.maximum(x, 0.0) + jnp.log1p(jnp.exp(-jnp.abs(x)))


def _tc1_body(x_ref, w_ref, ps_ref, pd_ref):
    p = jnp.dot(x_ref[...], w_ref[...], preferred_element_type=f32)
    ps_ref[...] = p[:, :H]
    pd_ref[...] = p[:, H:]


def _project(x, w_sd):
    return pl.pallas_call(
        _tc1_body,
        grid=(1,),
        in_specs=[
            pl.BlockSpec((N, DN), lambda i: (0, 0)),
            pl.BlockSpec((DN, 2 * H), lambda i: (0, 0)),
        ],
        out_specs=[
            pl.BlockSpec((N, H), lambda i: (0, 0)),
            pl.BlockSpec((N, H), lambda i: (0, 0)),
        ],
        out_shape=[
            jax.ShapeDtypeStruct((N, H), f32),
            jax.ShapeDtypeStruct((N, H), f32),
        ],
    )(x, w_sd)


_sc_mesh = plsc.VectorSubcoreMesh(core_axis_name="c", subcore_axis_name="s")


@functools.partial(
    pl.kernel,
    out_type=[
        jax.ShapeDtypeStruct((EC, CH, H), f32),
        jax.ShapeDtypeStruct((EC, CH, H), f32),
    ],
    mesh=_sc_mesh,
    scratch_types=[
        pltpu.VMEM((SLAB, CH), jnp.int32),
        pltpu.VMEM((SLAB, CH), jnp.int32),
        pltpu.VMEM((SLAB, CH, H), f32),
        pltpu.SemaphoreType.DMA,
    ],
    compiler_params=pltpu.CompilerParams(use_tc_tiling_on_sc=False),
)
def _sc_gather(ps_hbm, pd_hbm, src_hbm, dst_hbm, gs_hbm, gd_hbm,
               sidx, didx, rows, sem):
    wid = lax.axis_index("s") * 2 + lax.axis_index("c")

    def slab_body(sl, carry):
        row0 = wid * CPW + sl * SLAB
        pltpu.sync_copy(src_hbm.at[pl.ds(row0, SLAB)], sidx)
        descs = [
            pltpu.async_copy(ps_hbm.at[sidx.at[j]], rows.at[j], sem)
            for j in range(SLAB)
        ]
        for d in descs:
            d.wait()
        pltpu.sync_copy(rows, gs_hbm.at[pl.ds(row0, SLAB)])
        pltpu.sync_copy(dst_hbm.at[pl.ds(row0, SLAB)], didx)
        descs = [
            pltpu.async_copy(pd_hbm.at[didx.at[j]], rows.at[j], sem)
            for j in range(SLAB)
        ]
        for d in descs:
            d.wait()
        pltpu.sync_copy(rows, gd_hbm.at[pl.ds(row0, SLAB)])
        return carry

    lax.fori_loop(0, NSLAB, slab_body, 0)


def _tc2_body(gs_ref, gd_ref, ea_ref, g4_ref, wg4_ref, b1_ref, wd1_ref,
              wd2_ref, b2_ref, wd3_ref, b3_ref, he_ref, sum_ref):
    i = pl.program_id(0)
    c128 = jnp.dot(g4_ref[...], wg4_ref[...], preferred_element_type=f32) + b1_ref[...]
    z = (gs_ref[...] + gd_ref[...]
         + jnp.dot(ea_ref[...], wd1_ref[...], preferred_element_type=f32)
         + c128)
    h1 = _softplus(z)
    h2 = _softplus(jnp.dot(h1, wd2_ref[...], preferred_element_type=f32) + b2_ref[...])
    he = jnp.dot(h2, wd3_ref[...], preferred_element_type=f32) + b3_ref[...]
    he_ref[...] = he
    bsum = jnp.sum(he, axis=0, keepdims=True)

    @pl.when(i == 0)
    def _():
        sum_ref[...] = bsum

    @pl.when(i != 0)
    def _():
        sum_ref[...] = sum_ref[...] + bsum


def _edge_mlp(gs4, gd4, ea4, g4, wg4, b1_4, wd1, wd2, b2_4, wd3, b3_4):
    grid = (E4 // RB_E,)
    full = lambda shape: pl.BlockSpec(shape, lambda i: (0, 0))
    return pl.pallas_call(
        _tc2_body,
        grid=grid,
        in_specs=[
            pl.BlockSpec((RB_E, PK * H), lambda i: (i, 0)),
            pl.BlockSpec((RB_E, PK * H), lambda i: (i, 0)),
            pl.BlockSpec((RB_E, PK * DE), lambda i: (i, 0)),
            full((1, PK * DG)),
            full((PK * DG, PK * H)),
            full((1, PK * H)),
            full((PK * DE, PK * H)),
            full((PK * H, PK * H)),
            full((1, PK * H)),
            full((PK * H, PK * H)),
            full((1, PK * H)),
        ],
        out_specs=[
            pl.BlockSpec((RB_E, PK * H), lambda i: (i, 0)),
            pl.BlockSpec((1, PK * H), lambda i: (0, 0)),
        ],
        out_shape=[
            jax.ShapeDtypeStruct((E4, PK * H), f32),
            jax.ShapeDtypeStruct((1, PK * H), f32),
        ],
    )(gs4, gd4, ea4, g4, wg4, b1_4, wd1, wd2, b2_4, wd3, b3_4)


@functools.partial(
    pl.kernel,
    out_type=[
        jax.ShapeDtypeStruct((2, NP, H), f32),
        jax.ShapeDtypeStruct((2, NP, DE), f32),
    ],
    mesh=_sc_mesh,
    scratch_types=[
        pltpu.VMEM_SHARED((NP, H), f32),
        pltpu.VMEM_SHARED((NP, DE), f32),
        pltpu.VMEM((SLAB, CH), jnp.int32),
        pltpu.VMEM((SLAB, CH, H), f32),
        pltpu.VMEM((CH, DE), f32),
        pltpu.VMEM((NPT, H), f32),
        pltpu.VMEM((NPT, DE), f32),
        pltpu.SemaphoreType.DMA,
        pltpu.SemaphoreType.DMA,
    ],
    compiler_params=pltpu.CompilerParams(use_tc_tiling_on_sc=False),
)
def _sc_scatter(he_hbm, dst_hbm, z32_hbm, z16_hbm, ones_hbm, acc_hbm, cnt_hbm,
                sh_acc, sh_cnt, idx, rows, ones, stage, stage16, sem, sem2):
    cid = lax.axis_index("c")
    sid = lax.axis_index("s")
    wid = sid * 2 + cid

    pltpu.sync_copy(z32_hbm.at[pl.ds(sid * NPT, NPT)], stage)
    pltpu.sync_copy(stage, sh_acc.at[pl.ds(sid * NPT, NPT)])
    pltpu.sync_copy(z16_hbm.at[pl.ds(sid * NPT, NPT)], stage16)
    pltpu.sync_copy(stage16, sh_cnt.at[pl.ds(sid * NPT, NPT)])
    pltpu.sync_copy(ones_hbm, ones)
    plsc.subcore_barrier()

    def slab_body(sl, carry):
        row0 = wid * CPW + sl * SLAB
        pltpu.sync_copy(dst_hbm.at[pl.ds(row0, SLAB)], idx)
        pltpu.sync_copy(he_hbm.at[pl.ds(row0, SLAB)], rows)
        descs = [
            pltpu.async_copy(rows.at[j], sh_acc.at[idx.at[j]], sem, add=True)
            for j in range(SLAB)
        ]
        descs2 = [
            pltpu.async_copy(ones, sh_cnt.at[idx.at[j]], sem2, add=True)
            for j in range(SLAB)
        ]
        for d in descs:
            d.wait()
        for d in descs2:
            d.wait()
        return carry

    lax.fori_loop(0, NSLAB, slab_body, 0)
    plsc.subcore_barrier()

    pltpu.sync_copy(sh_acc.at[pl.ds(sid * NPT, NPT)], stage)
    pltpu.sync_copy(stage, acc_hbm.at[cid, pl.ds(sid * NPT, NPT)])
    pltpu.sync_copy(sh_cnt.at[pl.ds(sid * NPT, NPT)], stage16)
    pltpu.sync_copy(stage16, cnt_hbm.at[cid, pl.ds(sid * NPT, NPT)])


def _tc3_body(x_ref, acc_ref, cnt_ref, she_ref, g_ref,
              wnx_ref, wne_ref, wng_ref, bn1_ref, wn2_ref, bn2_ref,
              wn3_ref, bn3_ref,
              wg1e_ref, wg1n_ref, wg1g_ref, bg1_ref, wg2_ref, bg2_ref,
              wg3_ref, bg3_ref,
              hn_ref, hu_ref, ssum):
    i = pl.program_id(0)
    cnt = cnt_ref[0, :, 0:1] + cnt_ref[1, :, 0:1]
    e_mean = (acc_ref[0] + acc_ref[1]) / jnp.maximum(cnt, 1.0)
    grow = jnp.dot(g_ref[...], wng_ref[...], preferred_element_type=f32)
    h1 = _softplus(jnp.dot(x_ref[...], wnx_ref[...], preferred_element_type=f32)
                   + jnp.dot(e_mean, wne_ref[...], preferred_element_type=f32)
                   + grow + bn1_ref[...])
    h2 = _softplus(jnp.dot(h1, wn2_ref[...], preferred_element_type=f32) + bn2_ref[...])
    hn = jnp.dot(h2, wn3_ref[...], preferred_element_type=f32) + bn3_ref[...]
    hn_ref[...] = hn
    bsum = jnp.sum(hn, axis=0, keepdims=True)

    @pl.when(i == 0)
    def _():
        ssum[...] = bsum

    @pl.when(i != 0)
    def _():
        ssum[...] = ssum[...] + bsum

    e_mean_g = jnp.sum(she_ref[...], axis=0, keepdims=True) * (1.0 / E)
    n_mean_g = ssum[...] * (1.0 / N)
    zg = (jnp.dot(e_mean_g, wg1e_ref[...], preferred_element_type=f32)
          + jnp.dot(n_mean_g, wg1n_ref[...], preferred_element_type=f32)
          + jnp.dot(g_ref[...], wg1g_ref[...], preferred_element_type=f32)
          + bg1_ref[...])
    hg1 = _softplus(zg)
    hg2 = _softplus(jnp.dot(hg1, wg2_ref[...], preferred_element_type=f32) + bg2_ref[...])
    hu_ref[...] = jnp.dot(hg2, wg3_ref[...], preferred_element_type=f32) + bg3_ref[...]


def _node_global(x, acc, cnt, she, g, wnx, wne, wng, bn1, wn2, bn2, wn3, bn3,
                 wg1e, wg1n, wg1g, bg1, wg2, bg2, wg3, bg3):
    grid = (N // RB_N,)
    full = lambda shape: pl.BlockSpec(shape, lambda i: tuple(0 for _ in shape))
    return pl.pallas_call(
        _tc3_body,
        grid=grid,
        in_specs=[
            pl.BlockSpec((RB_N, DN), lambda i: (i, 0)),
            pl.BlockSpec((2, RB_N, H), lambda i: (0, i, 0)),
            pl.BlockSpec((2, RB_N, DE), lambda i: (0, i, 0)),
            full((PK, H)),
            full((1, DG)),
            full((DN, H)),
            full((H, H)),
            full((DG, H)),
            full((1, H)),
            full((H, H)),
            full((1, H)),
            full((H, H)),
            full((1, H)),
            full((H, DG)),
            full((H, DG)),
            full((DG, DG)),
            full((1, DG)),
            full((DG, DG)),
            full((1, DG)),
            full((DG, DG)),
            full((1, DG)),
        ],
        out_specs=[
            pl.BlockSpec((RB_N, H), lambda i: (i, 0)),
            pl.BlockSpec((1, DG), lambda i: (0, 0)),
        ],
        out_shape=[
            jax.ShapeDtypeStruct((N, H), f32),
            jax.ShapeDtypeStruct((1, DG), f32),
        ],
        scratch_shapes=[pltpu.VMEM((1, H), f32)],
    )(x, acc, cnt, she, g, wnx, wne, wng, bn1, wn2, bn2, wn3, bn3,
      wg1e, wg1n, wg1g, bg1, wg2, bg2, wg3, bg3)


def kernel(edge_index, x, edge_attr, global_feats, batch,
           We1, be1, We2, be2, We3, be3,
           Wn1, bn1, Wn2, bn2, Wn3, bn3,
           Wg1, bg1, Wg2, bg2, Wg3, bg3):
    w_s = We1[:DN]
    w_d = We1[DN:2 * DN]
    w_e = We1[2 * DN:2 * DN + DE]
    w_g = We1[2 * DN + DE:]
    eye = jnp.eye(PK, dtype=f32)
    wd1 = jnp.kron(eye, w_e)
    wg4 = jnp.kron(eye, w_g)
    wd2 = jnp.kron(eye, We2)
    wd3 = jnp.kron(eye, We3)
    b1_4 = jnp.tile(be1, PK)[None, :]
    b2_4 = jnp.tile(be2, PK)[None, :]
    b3_4 = jnp.tile(be3, PK)[None, :]
    g4 = jnp.tile(global_feats, (1, PK))

    src2d = edge_index[0].reshape(EC, CH)
    dst2d = edge_index[1].reshape(EC, CH)
    ea4 = edge_attr.reshape(E4, PK * DE)
    z32 = jnp.zeros((NP, H), f32)
    z16 = jnp.zeros((NP, DE), f32)
    ones16 = jnp.ones((CH, DE), f32)

    pre_s, pre_d = _project(x, jnp.concatenate([w_s, w_d], axis=1))

    gs, gd = _sc_gather(pre_s, pre_d, src2d, dst2d)

    he4, sum128 = _edge_mlp(gs.reshape(E4, PK * H), gd.reshape(E4, PK * H),
                            ea4, g4, wg4, b1_4, wd1, wd2, b2_4, wd3, b3_4)

    acc, cnt = _sc_scatter(he4.reshape(EC, CH, H), dst2d, z32, z16, ones16)
    h_e = he4.reshape(E, H)

    h_n, h_u = _node_global(
        x, acc, cnt, sum128.reshape(PK, H), global_feats,
        Wn1[:DN], Wn1[DN:DN + H], Wn1[DN + H:], bn1[None, :],
        Wn2, bn2[None, :], Wn3, bn3[None, :],
        Wg1[:H], Wg1[H:2 * H], Wg1[2 * H:], bg1[None, :],
        Wg2, bg2[None, :], Wg3, bg3[None, :])

    return (h_e, h_n, h_u)
